# baseline (speedup 1.0000x reference)
import functools

import jax
import jax.numpy as jnp
from jax import lax
from jax.experimental import pallas as pl
from jax.experimental.pallas import tpu as pltpu
from jax.experimental.pallas import tpu_sc as plsc

MAXP = 128
D = 128
K = 8
B = 16
N = 2048
NT = B * N
NW = 32
TPW = NT // NW
OUT_CHUNK = 128
GROUPS = OUT_CHUNK // 16

OFF_Y = MAXP
OFF_W = 2 * MAXP
OFF_H = 2 * MAXP + 257
PR = 776


def _proj_body(tbl_ref, w_ref, b_ref, out_ref):
    proj = lax.dot_general(
        tbl_ref[...], w_ref[...], (((1,), (1,)), ((), ())),
        preferred_element_type=jnp.float32,
        precision=lax.Precision.HIGHEST)
    rows = lax.broadcasted_iota(jnp.int32, (PR, D), 0)
    out_ref[...] = proj + jnp.where(rows < MAXP, b_ref[...] * 0.25, 0.0)


_proj = pl.pallas_call(
    _proj_body,
    out_shape=jax.ShapeDtypeStruct((PR, D), jnp.float32),
)

_mesh = plsc.VectorSubcoreMesh(core_axis_name="c", subcore_axis_name="s")


HW = D // 2


@functools.partial(
    pl.kernel,
    out_type=jax.ShapeDtypeStruct((NT * HW,), jnp.int32),
    mesh=_mesh,
    scratch_types=[
        pltpu.VMEM((PR * HW,), jnp.int32),
        pltpu.VMEM((K, TPW), jnp.float32),
        pltpu.VMEM((K, 16), jnp.float32),
        pltpu.VMEM((OUT_CHUNK * HW,), jnp.int32),
    ],
    compiler_params=pltpu.CompilerParams(needs_layout_passes=False),
)
def _sc_gather(p_hbm, boxes_hbm, mult_hbm, out_hbm, p_v, box_v, mult_v, out_v):
    wid = lax.axis_index("s") * 2 + lax.axis_index("c")
    pltpu.sync_copy(p_hbm, p_v)
    pltpu.sync_copy(boxes_hbm.at[wid], box_v)
    pltpu.sync_copy(mult_hbm.at[wid], mult_v)
    lane = lax.iota(jnp.int32, 16)
    mults = [mult_v[k, :] for k in range(K)]

    def chunk_body(ci, carry):
        def group_body(gi, carry2):
            tok0 = ci * OUT_CHUNK + gi * 16
            idx = []
            for k in range(K):
                f = box_v[k, pl.ds(tok0, 16)] * mults[k]
                f = jnp.maximum(jnp.minimum(f, 127.0), 0.0)
                fi = f.astype(jnp.int32)
                fi = fi - (fi.astype(jnp.float32) > f).astype(jnp.int32)
                idx.append(fi)
            rows = []
            for i in range(4):
                rows.append(idx[2 * i])
                rows.append(idx[2 * i + 1] + OFF_Y)
            for i in range(4):
                rows.append(idx[(2 * i + 2) % 8] - idx[2 * i] + (OFF_W + MAXP))
            for i in range(4):
                rows.append(idx[(2 * i + 3) % 8] - idx[2 * i + 1] + (OFF_H + MAXP))
            rowsf = [r * HW for r in rows]
            zero16 = jnp.zeros((16,), jnp.int32)

            def tok_body(th, c2):
                for dt in range(2):
                    t = th * 2 + dt
                    tsel = zero16 + t
                    base = [jnp.take(rowsf[j], tsel) + lane
                            for j in range(16)]
                    obase = (gi * 16 + t) * HW
                    for c in range(0, HW, 16):
                        v = [plsc.bitcast(
                                plsc.load_gather(p_v, [base[j] + c]),
                                jnp.bfloat16)
                             for j in range(16)]
                        s = [(v[2 * j] + v[2 * j + 1]) for j in range(8)]
                        u = [(s[2 * j] + s[2 * j + 1]) for j in range(4)]
                        acc = (u[0] + u[1]) + (u[2] + u[3])
                        out_v[pl.ds(obase + c, 16)] = plsc.bitcast(
                            acc, jnp.int32)
                return c2
            lax.fori_loop(0, 8, tok_body, 0)
            return carry2
        lax.fori_loop(0, GROUPS, group_body, 0)
        pltpu.sync_copy(
            out_v,
            out_hbm.at[pl.ds((wid * TPW + ci * OUT_CHUNK) * HW,
                             OUT_CHUNK * HW)])
        return carry
    lax.fori_loop(0, TPW // OUT_CHUNK, chunk_body, 0)


def kernel(boxes, img_shapes, x_table, y_table, w_table, h_table, W, b):
    ptbl = jnp.concatenate([x_table, y_table, w_table, h_table], axis=0)
    ptbl = jnp.pad(ptbl, ((0, PR - ptbl.shape[0]), (0, 0)))
    p = _proj(ptbl, W, b.reshape(1, D))

    bt = boxes.reshape(NT, K).T.reshape(K, NW, TPW).transpose(1, 0, 2)

    w_sc = img_shapes[:, 1].astype(jnp.float32)
    h_sc = img_shapes[:, 0].astype(jnp.float32)
    even = (jnp.arange(K) % 2 == 0)
    mult_b = MAXP / jnp.where(even[None, :], w_sc[:, None], h_sc[:, None])
    mult_w = jnp.repeat(mult_b, NW // B, axis=0)
    mult16 = jnp.broadcast_to(mult_w[:, :, None], (NW, K, 16))

    pb = lax.bitcast_convert_type(
        p.astype(jnp.bfloat16).reshape(PR, HW, 2), jnp.int32)
    out_packed = _sc_gather(pb.reshape(PR * HW), bt, mult16)
    out_bf = lax.bitcast_convert_type(
        out_packed.reshape(NT, HW), jnp.bfloat16)
    return out_bf.reshape(B, N, D).astype(jnp.float32)


# speedup vs baseline: 1.2235x; 1.2235x over previous
import functools

import jax
import jax.numpy as jnp
from jax import lax
from jax.experimental import pallas as pl
from jax.experimental.pallas import tpu as pltpu
from jax.experimental.pallas import tpu_sc as plsc

MAXP = 128
D = 128
K = 8
B = 16
N = 2048
NT = B * N
NW = 32
TPW = NT // NW
OUT_CHUNK = 128
GROUPS = OUT_CHUNK // 16

OFF_Y = MAXP
OFF_W = 2 * MAXP
OFF_H = 2 * MAXP + 257
PR = 776


def _proj_body(tbl_ref, w_ref, b_ref, out_ref):
    proj = lax.dot_general(
        tbl_ref[...], w_ref[...], (((1,), (1,)), ((), ())),
        preferred_element_type=jnp.float32,
        precision=lax.Precision.HIGHEST)
    rows = lax.broadcasted_iota(jnp.int32, (PR, D), 0)
    out_ref[...] = proj + jnp.where(rows < MAXP, b_ref[...] * 0.25, 0.0)


_proj = pl.pallas_call(
    _proj_body,
    out_shape=jax.ShapeDtypeStruct((PR, D), jnp.float32),
)

_mesh = plsc.VectorSubcoreMesh(core_axis_name="c", subcore_axis_name="s")


@functools.partial(
    pl.kernel,
    out_type=jax.ShapeDtypeStruct((NT * D,), jnp.float32),
    mesh=_mesh,
    scratch_types=[
        pltpu.VMEM((PR * D,), jnp.float32),
        pltpu.VMEM((K, TPW), jnp.float32),
        pltpu.VMEM((K, 16), jnp.float32),
        pltpu.VMEM((OUT_CHUNK * D,), jnp.float32),
    ],
    compiler_params=pltpu.CompilerParams(needs_layout_passes=False),
)
def _sc_gather(p_hbm, boxes_hbm, mult_hbm, out_hbm, p_v, box_v, mult_v, out_v):
    wid = lax.axis_index("s") * 2 + lax.axis_index("c")
    pltpu.sync_copy(p_hbm, p_v)
    pltpu.sync_copy(boxes_hbm.at[wid], box_v)
    pltpu.sync_copy(mult_hbm.at[wid], mult_v)
    lane = lax.iota(jnp.int32, 16)
    mults = [mult_v[k, :] for k in range(K)]

    def chunk_body(ci, carry):
        def group_body(gi, carry2):
            tok0 = ci * OUT_CHUNK + gi * 16
            idx = []
            for k in range(K):
                f = box_v[k, pl.ds(tok0, 16)] * mults[k]
                f = jnp.maximum(jnp.minimum(f, 127.0), 0.0)
                fi = f.astype(jnp.int32)
                fi = fi - (fi.astype(jnp.float32) > f).astype(jnp.int32)
                idx.append(fi)
            rows = []
            for i in range(4):
                rows.append(idx[2 * i])
                rows.append(idx[2 * i + 1] + OFF_Y)
            for i in range(4):
                rows.append(idx[(2 * i + 2) % 8] - idx[2 * i] + (OFF_W + MAXP))
            for i in range(4):
                rows.append(idx[(2 * i + 3) % 8] - idx[2 * i + 1] + (OFF_H + MAXP))
            rowsf = [r * D for r in rows]
            zero16 = jnp.zeros((16,), jnp.int32)

            def tok_body(t, c2):
                tsel = zero16 + t
                base = [jnp.take(rowsf[j], tsel) + lane
                        for j in range(16)]
                obase = (gi * 16 + t) * D
                for c in range(0, D, 32):
                    va = [plsc.load_gather(p_v, [base[j] + c])
                          for j in range(16)]
                    vb = [plsc.load_gather(p_v, [base[j] + (c + 16)])
                          for j in range(16)]
                    sa = [(va[2 * j] + va[2 * j + 1]) for j in range(8)]
                    sb = [(vb[2 * j] + vb[2 * j + 1]) for j in range(8)]
                    ua = [(sa[2 * j] + sa[2 * j + 1]) for j in range(4)]
                    ub = [(sb[2 * j] + sb[2 * j + 1]) for j in range(4)]
                    out_v[pl.ds(obase + c, 16)] = (
                        (ua[0] + ua[1]) + (ua[2] + ua[3]))
                    out_v[pl.ds(obase + c + 16, 16)] = (
                        (ub[0] + ub[1]) + (ub[2] + ub[3]))
                return c2
            lax.fori_loop(0, 16, tok_body, 0)
            return carry2
        lax.fori_loop(0, GROUPS, group_body, 0)
        pltpu.sync_copy(
            out_v,
            out_hbm.at[pl.ds((wid * TPW + ci * OUT_CHUNK) * D, OUT_CHUNK * D)])
        return carry
    lax.fori_loop(0, TPW // OUT_CHUNK, chunk_body, 0)


def kernel(boxes, img_shapes, x_table, y_table, w_table, h_table, W, b):
    ptbl = jnp.concatenate([x_table, y_table, w_table, h_table], axis=0)
    ptbl = jnp.pad(ptbl, ((0, PR - ptbl.shape[0]), (0, 0)))
    p = _proj(ptbl, W, b.reshape(1, D))

    bt = boxes.reshape(NT, K).T.reshape(K, NW, TPW).transpose(1, 0, 2)

    w_sc = img_shapes[:, 1].astype(jnp.float32)
    h_sc = img_shapes[:, 0].astype(jnp.float32)
    even = (jnp.arange(K) % 2 == 0)
    mult_b = MAXP / jnp.where(even[None, :], w_sc[:, None], h_sc[:, None])
    mult_w = jnp.repeat(mult_b, NW // B, axis=0)
    mult16 = jnp.broadcast_to(mult_w[:, :, None], (NW, K, 16))

    out_flat = _sc_gather(p.reshape(PR * D), bt, mult16)
    return out_flat.reshape(B, N, D)


# speedup vs baseline: 1.2629x; 1.0322x over previous
import functools

import jax
import jax.numpy as jnp
from jax import lax
from jax.experimental import pallas as pl
from jax.experimental.pallas import tpu as pltpu
from jax.experimental.pallas import tpu_sc as plsc

MAXP = 128
D = 128
K = 8
B = 16
N = 2048
NT = B * N
NW = 32
TPW = NT // NW
OUT_CHUNK = 128
GROUPS = OUT_CHUNK // 16

OFF_Y = MAXP
OFF_W = 2 * MAXP
OFF_H = 2 * MAXP + 257
PR = 776


def _proj_body(tbl_ref, w_ref, b_ref, out_ref):
    proj = lax.dot_general(
        tbl_ref[...], w_ref[...], (((1,), (1,)), ((), ())),
        preferred_element_type=jnp.float32,
        precision=lax.Precision.HIGHEST)
    rows = lax.broadcasted_iota(jnp.int32, (PR, D), 0)
    out_ref[...] = proj + jnp.where(rows < MAXP, b_ref[...] * 0.25, 0.0)


_proj = pl.pallas_call(
    _proj_body,
    out_shape=jax.ShapeDtypeStruct((PR, D), jnp.float32),
)

_mesh = plsc.VectorSubcoreMesh(core_axis_name="c", subcore_axis_name="s")


@functools.partial(
    pl.kernel,
    out_type=jax.ShapeDtypeStruct((NT * D,), jnp.float32),
    mesh=_mesh,
    scratch_types=[
        pltpu.VMEM((PR * D,), jnp.float32),
        pltpu.VMEM((K, TPW), jnp.float32),
        pltpu.VMEM((K, 16), jnp.float32),
        pltpu.VMEM((OUT_CHUNK * D,), jnp.float32),
    ],
    compiler_params=pltpu.CompilerParams(needs_layout_passes=False),
)
def _sc_gather(p_hbm, boxes_hbm, mult_hbm, out_hbm, p_v, box_v, mult_v, out_v):
    wid = lax.axis_index("s") * 2 + lax.axis_index("c")
    pltpu.sync_copy(p_hbm, p_v)
    pltpu.sync_copy(boxes_hbm.at[wid], box_v)
    pltpu.sync_copy(mult_hbm.at[wid], mult_v)
    lane = lax.iota(jnp.int32, 16)
    mults = [mult_v[k, :] for k in range(K)]

    def chunk_body(ci, carry):
        def group_body(gi, carry2):
            tok0 = ci * OUT_CHUNK + gi * 16
            idx = []
            for k in range(K):
                f = box_v[k, pl.ds(tok0, 16)] * mults[k]
                f = jnp.maximum(jnp.minimum(f, 127.0), 0.0)
                fi = f.astype(jnp.int32)
                fi = fi - (fi.astype(jnp.float32) > f).astype(jnp.int32)
                idx.append(fi)
            rows = []
            for i in range(4):
                rows.append(idx[2 * i])
                rows.append(idx[2 * i + 1] + OFF_Y)
            for i in range(4):
                rows.append(idx[(2 * i + 2) % 8] - idx[2 * i] + (OFF_W + MAXP))
            for i in range(4):
                rows.append(idx[(2 * i + 3) % 8] - idx[2 * i + 1] + (OFF_H + MAXP))
            rowsf = [r * D for r in rows]
            zero16 = jnp.zeros((16,), jnp.int32)

            def tok_body(th, c2):
              for dt in range(2):
                t = th * 2 + dt
                tsel = zero16 + t
                base = [jnp.take(rowsf[j], tsel) + lane
                        for j in range(16)]
                obase = (gi * 16 + t) * D
                for c in range(0, D, 32):
                    va = [plsc.load_gather(p_v, [base[j] + c])
                          for j in range(16)]
                    vb = [plsc.load_gather(p_v, [base[j] + (c + 16)])
                          for j in range(16)]
                    sa = [(va[2 * j] + va[2 * j + 1]) for j in range(8)]
                    sb = [(vb[2 * j] + vb[2 * j + 1]) for j in range(8)]
                    ua = [(sa[2 * j] + sa[2 * j + 1]) for j in range(4)]
                    ub = [(sb[2 * j] + sb[2 * j + 1]) for j in range(4)]
                    out_v[pl.ds(obase + c, 16)] = (
                        (ua[0] + ua[1]) + (ua[2] + ua[3]))
                    out_v[pl.ds(obase + c + 16, 16)] = (
                        (ub[0] + ub[1]) + (ub[2] + ub[3]))
              return c2
            lax.fori_loop(0, 8, tok_body, 0)
            return carry2
        lax.fori_loop(0, GROUPS, group_body, 0)
        pltpu.sync_copy(
            out_v,
            out_hbm.at[pl.ds((wid * TPW + ci * OUT_CHUNK) * D, OUT_CHUNK * D)])
        return carry
    lax.fori_loop(0, TPW // OUT_CHUNK, chunk_body, 0)


def kernel(boxes, img_shapes, x_table, y_table, w_table, h_table, W, b):
    ptbl = jnp.concatenate([x_table, y_table, w_table, h_table], axis=0)
    ptbl = jnp.pad(ptbl, ((0, PR - ptbl.shape[0]), (0, 0)))
    p = _proj(ptbl, W, b.reshape(1, D))

    bt = boxes.reshape(NT, K).T.reshape(K, NW, TPW).transpose(1, 0, 2)

    w_sc = img_shapes[:, 1].astype(jnp.float32)
    h_sc = img_shapes[:, 0].astype(jnp.float32)
    even = (jnp.arange(K) % 2 == 0)
    mult_b = MAXP / jnp.where(even[None, :], w_sc[:, None], h_sc[:, None])
    mult_w = jnp.repeat(mult_b, NW // B, axis=0)
    mult16 = jnp.broadcast_to(mult_w[:, :, None], (NW, K, 16))

    out_flat = _sc_gather(p.reshape(PR * D), bt, mult16)
    return out_flat.reshape(B, N, D)


# speedup vs baseline: 1.3113x; 1.0383x over previous
import functools

import jax
import jax.numpy as jnp
from jax import lax
from jax.experimental import pallas as pl
from jax.experimental.pallas import tpu as pltpu
from jax.experimental.pallas import tpu_sc as plsc

MAXP = 128
D = 128
K = 8
B = 16
N = 2048
NT = B * N
NW = 32
TPW = NT // NW
OUT_CHUNK = 128
GROUPS = OUT_CHUNK // 16

OFF_Y = MAXP
OFF_W = 2 * MAXP
OFF_H = 2 * MAXP + 257
PR = 776


def _proj_body(tbl_ref, w_ref, b_ref, out_ref):
    proj = lax.dot_general(
        tbl_ref[...], w_ref[...], (((1,), (1,)), ((), ())),
        preferred_element_type=jnp.float32,
        precision=lax.Precision.HIGHEST)
    rows = lax.broadcasted_iota(jnp.int32, (PR, D), 0)
    out_ref[...] = proj + jnp.where(rows < MAXP, b_ref[...] * 0.25, 0.0)


_proj = pl.pallas_call(
    _proj_body,
    out_shape=jax.ShapeDtypeStruct((PR, D), jnp.float32),
)

_mesh = plsc.VectorSubcoreMesh(core_axis_name="c", subcore_axis_name="s")


@functools.partial(
    pl.kernel,
    out_type=jax.ShapeDtypeStruct((NT * D,), jnp.float32),
    mesh=_mesh,
    scratch_types=[
        pltpu.VMEM((PR * D,), jnp.float32),
        pltpu.VMEM((K, TPW), jnp.float32),
        pltpu.VMEM((K, 16), jnp.float32),
        pltpu.VMEM((OUT_CHUNK * D,), jnp.float32),
    ],
    compiler_params=pltpu.CompilerParams(needs_layout_passes=False),
)
def _sc_gather(p_hbm, boxes_hbm, mult_hbm, out_hbm, p_v, box_v, mult_v, out_v):
    wid = lax.axis_index("s") * 2 + lax.axis_index("c")
    pltpu.sync_copy(p_hbm, p_v)
    pltpu.sync_copy(boxes_hbm.at[wid], box_v)
    pltpu.sync_copy(mult_hbm.at[wid], mult_v)
    lane = lax.iota(jnp.int32, 16)
    mults = [mult_v[k, :] for k in range(K)]

    def chunk_body(ci, carry):
        def group_body(gi, carry2):
            tok0 = ci * OUT_CHUNK + gi * 16
            idx = []
            for k in range(K):
                f = box_v[k, pl.ds(tok0, 16)] * mults[k]
                f = jnp.maximum(jnp.minimum(f, 127.0), 0.0)
                fi = f.astype(jnp.int32)
                fi = fi - (fi.astype(jnp.float32) > f).astype(jnp.int32)
                idx.append(fi)
            rows = []
            for i in range(4):
                rows.append(idx[2 * i])
                rows.append(idx[2 * i + 1] + OFF_Y)
            for i in range(4):
                rows.append(idx[(2 * i + 2) % 8] - idx[2 * i] + (OFF_W + MAXP))
            for i in range(4):
                rows.append(idx[(2 * i + 3) % 8] - idx[2 * i + 1] + (OFF_H + MAXP))
            rowsf = [r * D for r in rows]
            zero16 = jnp.zeros((16,), jnp.int32)

            def tok_body(th, c2):
              for dt in range(2):
                t = th * 2 + dt
                tsel = zero16 + t
                base = [jnp.take(rowsf[j], tsel) + lane
                        for j in range(16)]
                obase = (gi * 16 + t) * D
                for c in range(0, D, 32):
                    pa = p_v.at[pl.ds(c, PR * D - c)]
                    pb = p_v.at[pl.ds(c + 16, PR * D - c - 16)]
                    va = [plsc.load_gather(pa, [base[j]])
                          for j in range(16)]
                    vb = [plsc.load_gather(pb, [base[j]])
                          for j in range(16)]
                    sa = [(va[2 * j] + va[2 * j + 1]) for j in range(8)]
                    sb = [(vb[2 * j] + vb[2 * j + 1]) for j in range(8)]
                    ua = [(sa[2 * j] + sa[2 * j + 1]) for j in range(4)]
                    ub = [(sb[2 * j] + sb[2 * j + 1]) for j in range(4)]
                    out_v[pl.ds(obase + c, 16)] = (
                        (ua[0] + ua[1]) + (ua[2] + ua[3]))
                    out_v[pl.ds(obase + c + 16, 16)] = (
                        (ub[0] + ub[1]) + (ub[2] + ub[3]))
              return c2
            lax.fori_loop(0, 8, tok_body, 0)
            return carry2
        lax.fori_loop(0, GROUPS, group_body, 0)
        pltpu.sync_copy(
            out_v,
            out_hbm.at[pl.ds((wid * TPW + ci * OUT_CHUNK) * D, OUT_CHUNK * D)])
        return carry
    lax.fori_loop(0, TPW // OUT_CHUNK, chunk_body, 0)


def kernel(boxes, img_shapes, x_table, y_table, w_table, h_table, W, b):
    ptbl = jnp.concatenate([x_table, y_table, w_table, h_table], axis=0)
    ptbl = jnp.pad(ptbl, ((0, PR - ptbl.shape[0]), (0, 0)))
    p = _proj(ptbl, W, b.reshape(1, D))

    bt = boxes.reshape(NT, K).T.reshape(K, NW, TPW).transpose(1, 0, 2)

    w_sc = img_shapes[:, 1].astype(jnp.float32)
    h_sc = img_shapes[:, 0].astype(jnp.float32)
    even = (jnp.arange(K) % 2 == 0)
    mult_b = MAXP / jnp.where(even[None, :], w_sc[:, None], h_sc[:, None])
    mult_w = jnp.repeat(mult_b, NW // B, axis=0)
    mult16 = jnp.broadcast_to(mult_w[:, :, None], (NW, K, 16))

    out_flat = _sc_gather(p.reshape(PR * D), bt, mult16)
    return out_flat.reshape(B, N, D)
